# baseline (speedup 1.0000x reference)
import functools

import jax
import jax.numpy as jnp
from jax import lax
from jax.experimental import pallas as pl
from jax.experimental.pallas import tpu as pltpu
from jax.experimental.pallas import tpu_sc as plsc

T = 2048
H = 1024
E = 8
F = 1024
NSH = 2
A = 2 * T
TILE = 256
AMAX = A + E * TILE
NT = AMAX // TILE

NC = 2
NS = 16
NW = NC * NS

_DN = (((1,), (1,)), ((), ()))


def _router_kernel(x_ref, gate_ref, dest_ref, meta_ref, w0_ref, w1_ref):
    x = x_ref[...]
    logits = lax.dot_general(x, gate_ref[...], _DN,
                             preferred_element_type=jnp.float32)
    mx = jnp.max(logits, axis=1, keepdims=True)
    p = jnp.exp(logits - mx)
    p = p / jnp.sum(p, axis=1, keepdims=True)
    cols = lax.broadcasted_iota(jnp.int32, p.shape, 1)
    i1 = jnp.argmax(p, axis=1)
    m1 = jnp.max(p, axis=1)
    oh1 = cols == i1[:, None]
    p2 = jnp.where(oh1, -1.0, p)
    i2 = jnp.argmax(p2, axis=1)
    m2 = jnp.max(p2, axis=1)
    denom = m1 + m2 + 1e-20
    w0_ref[...] = jnp.broadcast_to((m1 / denom)[:, None], (T, 16))
    w1_ref[...] = jnp.broadcast_to((m2 / denom)[:, None], (T, 16))

    ohA = (cols == i1[:, None]).astype(jnp.float32)
    ohB = (cols == i2[:, None]).astype(jnp.float32)
    counts = jnp.sum(ohA, axis=0) + jnp.sum(ohB, axis=0)
    pc = jnp.floor((counts + (TILE - 1)) * (1.0 / TILE)) * TILE
    tri8 = (lax.broadcasted_iota(jnp.int32, (E, E), 0)
            > lax.broadcasted_iota(jnp.int32, (E, E), 1)).astype(jnp.float32)
    ps = lax.dot_general(pc[None, :], tri8, _DN,
                         preferred_element_type=jnp.float32)[0]
    total = jnp.sum(pc)
    na = total * (1.0 / TILE)

    ti = lax.broadcasted_iota(jnp.int32, (128, E), 0).astype(jnp.float32)
    ti = jnp.minimum(ti, na - 1.0) * float(TILE)
    te = jnp.sum((ps[None, :] <= ti).astype(jnp.float32), axis=1) - 1.0
    lane = lax.broadcasted_iota(jnp.int32, (128,), 0)
    meta = jnp.where(lane == NT, na, te)
    meta_ref[...] = meta.astype(jnp.int32)[None, :]

    CH = T // 2
    rows = lax.broadcasted_iota(jnp.int32, (CH, CH), 0)
    colsq = lax.broadcasted_iota(jnp.int32, (CH, CH), 1)
    lt = (rows >= colsq).astype(jnp.float32)
    carry = jnp.zeros((E,), jnp.float32)
    for c in range(4):
        oh = (ohA, ohA, ohB, ohB)[c]
        oh = oh[(c % 2) * CH:(c % 2) * CH + CH]
        cum = lax.dot_general(lt, oh, (((1,), (0,)), ((), ())),
                              preferred_element_type=jnp.float32)
        cum = cum + carry[None, :]
        carry = carry + jnp.sum(oh, axis=0)
        rank = jnp.sum(oh * cum, axis=1) - 1.0
        start = jnp.sum(oh * ps[None, :], axis=1)
        dest_ref[c, :] = (start + rank).astype(jnp.int32)


def _run_router(x, gate_weight):
    return pl.pallas_call(
        _router_kernel,
        out_shape=(
            jax.ShapeDtypeStruct((4, T // 2), jnp.int32),
            jax.ShapeDtypeStruct((1, 128), jnp.int32),
            jax.ShapeDtypeStruct((T, 16), jnp.float32),
            jax.ShapeDtypeStruct((T, 16), jnp.float32),
        ),
    )(x, gate_weight)


_DCH = 32
_DNCH = (A // NW) // _DCH


def _dispatch_body(x_hbm, dest_hbm, xs_hbm, idx_v, rows_v,
                   si0, si1, so0, so1):
    wid = lax.axis_index("s") * NC + lax.axis_index("c")
    apw = A // NW
    base = wid * apw
    tok = jnp.where(base >= T, base - T, base)
    sin = (si0, si1)
    sout = (so0, so1)

    def issue_in(c):
        b = c % 2
        off = c * _DCH
        ci = pltpu.async_copy(dest_hbm.at[pl.ds(base + off, _DCH)],
                              idx_v.at[b], sin[b])
        cr = pltpu.async_copy(x_hbm.at[pl.ds(tok + off, _DCH)],
                              rows_v.at[b], sin[b])
        return ci, cr

    cin = [issue_in(0), issue_in(1)]
    for c in range(_DNCH):
        b = c % 2
        ci, cr = cin[b]
        ci.wait()
        cr.wait()
        cs = pltpu.async_copy(rows_v.at[b], xs_hbm.at[idx_v.at[b]], sout[b])
        if c + 2 < _DNCH:
            cs.wait()
            cin[b] = issue_in(c + 2)
        else:
            cs.wait()


def _run_dispatch(x, dest):
    kern = functools.partial(
        pl.kernel,
        mesh=plsc.VectorSubcoreMesh(core_axis_name="c", subcore_axis_name="s"),
        out_type=jax.ShapeDtypeStruct((AMAX, H), jnp.float32),
        scratch_types=[
            pltpu.VMEM((2, _DCH), jnp.int32),
            pltpu.VMEM((2, _DCH, H), jnp.float32),
            pltpu.SemaphoreType.DMA,
            pltpu.SemaphoreType.DMA,
            pltpu.SemaphoreType.DMA,
            pltpu.SemaphoreType.DMA,
        ],
    )(_dispatch_body)
    return kern(x, dest)


def _grouped_kernel(tile_e_ref, na_ref, xs_ref, wg_ref, wu_ref, wd_ref,
                    ys_ref):
    i = pl.program_id(0)

    @pl.when(i < na_ref[0])
    def _():
        x = xs_ref[...]
        g = lax.dot_general(x, wg_ref[0].astype(jnp.bfloat16), _DN,
                            preferred_element_type=jnp.float32)
        u = lax.dot_general(x, wu_ref[0].astype(jnp.bfloat16), _DN,
                            preferred_element_type=jnp.float32)
        h = (g * jax.nn.sigmoid(g)) * u
        ys_ref[...] = lax.dot_general(h.astype(jnp.bfloat16),
                                      wd_ref[0].astype(jnp.bfloat16), _DN,
                                      preferred_element_type=jnp.float32)


def _run_grouped(tile_e, na, xs, Wg, Wu, Wd):
    grid_spec = pltpu.PrefetchScalarGridSpec(
        num_scalar_prefetch=2,
        grid=(NT,),
        in_specs=[
            pl.BlockSpec((TILE, H),
                         lambda i, te, na: (jnp.minimum(i, na[0] - 1), 0)),
            pl.BlockSpec((1, F, H), lambda i, te, na: (te[i], 0, 0)),
            pl.BlockSpec((1, F, H), lambda i, te, na: (te[i], 0, 0)),
            pl.BlockSpec((1, H, F), lambda i, te, na: (te[i], 0, 0)),
        ],
        out_specs=pl.BlockSpec(
            (TILE, H), lambda i, te, na: (jnp.minimum(i, na[0] - 1), 0)),
    )
    return pl.pallas_call(
        _grouped_kernel,
        grid_spec=grid_spec,
        out_shape=jax.ShapeDtypeStruct((AMAX, H), jnp.float32),
    )(tile_e, na, xs, Wg, Wu, Wd)


_FSH = 512
_NFS = (F * NSH) // _FSH


def _shared_kernel(x_ref, wsg_ref, wsu_ref, wsd_ref, out_ref):
    c = pl.program_id(0)
    x = x_ref[...].astype(jnp.bfloat16)
    g = lax.dot_general(x, wsg_ref[...].astype(jnp.bfloat16), _DN,
                        preferred_element_type=jnp.float32)
    u = lax.dot_general(x, wsu_ref[...].astype(jnp.bfloat16), _DN,
                        preferred_element_type=jnp.float32)
    h = (g * jax.nn.sigmoid(g)) * u
    y = lax.dot_general(h.astype(jnp.bfloat16),
                        wsd_ref[...].astype(jnp.bfloat16), _DN,
                        preferred_element_type=jnp.float32)

    @pl.when(c == 0)
    def _():
        out_ref[...] = y

    @pl.when(c > 0)
    def _():
        out_ref[...] += y


def _run_shared(x, Wsg, Wsu, Wsd):
    return pl.pallas_call(
        _shared_kernel,
        grid=(_NFS,),
        in_specs=[
            pl.BlockSpec((T, H), lambda c: (0, 0)),
            pl.BlockSpec((_FSH, H), lambda c: (c, 0)),
            pl.BlockSpec((_FSH, H), lambda c: (c, 0)),
            pl.BlockSpec((H, _FSH), lambda c: (0, c)),
        ],
        out_specs=pl.BlockSpec((T, H), lambda c: (0, 0)),
        out_shape=jax.ShapeDtypeStruct((T, H), jnp.float32),
    )(x, Wsg, Wsu, Wsd)


_CCH = 8
_CNCH = (T // NW) // _CCH


def _combine_body(ys_hbm, sh_hbm, p0_hbm, p1_hbm, w0_hbm, w1_hbm, y_hbm,
                  p0_v, p1_v, r0_v, r1_v, sh_v, w0_v, w1_v, out_v,
                  sx0, sx1, sg0, sg1, so0, so1):
    wid = lax.axis_index("s") * NC + lax.axis_index("c")
    tpw = T // NW
    base = wid * tpw
    sidx = (sx0, sx1)
    sgat = (sg0, sg1)
    sout = (so0, so1)

    def issue_idx(c):
        b = c % 2
        off = base + c * _CCH
        return (pltpu.async_copy(p0_hbm.at[pl.ds(off, _CCH)],
                                 p0_v.at[b], sidx[b]),
                pltpu.async_copy(p1_hbm.at[pl.ds(off, _CCH)],
                                 p1_v.at[b], sidx[b]))

    def issue_gather(c):
        b = c % 2
        off = base + c * _CCH
        return (pltpu.async_copy(ys_hbm.at[p0_v.at[b]], r0_v.at[b], sgat[b]),
                pltpu.async_copy(ys_hbm.at[p1_v.at[b]], r1_v.at[b], sgat[b]),
                pltpu.async_copy(sh_hbm.at[pl.ds(off, _CCH)],
                                 sh_v.at[b], sgat[b]),
                pltpu.async_copy(w0_hbm.at[pl.ds(off, _CCH)],
                                 w0_v.at[b], sgat[b]),
                pltpu.async_copy(w1_hbm.at[pl.ds(off, _CCH)],
                                 w1_v.at[b], sgat[b]))

    def wait_all(cps):
        for cp in cps:
            cp.wait()

    cidx = [issue_idx(0), None]
    wait_all(cidx[0])
    cgat = [issue_gather(0), None]
    cidx[1] = issue_idx(1)
    cout = [None, None]
    for c in range(_CNCH):
        b = c % 2
        nb = (c + 1) % 2
        if c + 1 < _CNCH:
            wait_all(cidx[nb])
            cgat[nb] = issue_gather(c + 1)
            if c + 2 < _CNCH:
                cidx[b] = None
        wait_all(cgat[b])
        if cout[b] is not None:
            wait_all(cout[b])

        def body(j, _):
            w0 = w0_v[b, j, :]
            w1 = w1_v[b, j, :]
            for cc in range(H // 16):
                s = pl.ds(cc * 16, 16)
                out_v[b, j, s] = (sh_v[b, j, s] + w0 * r0_v[b, j, s]
                                  + w1 * r1_v[b, j, s])
            return 0

        lax.fori_loop(0, _CCH, body, 0)
        off = base + c * _CCH
        cout[b] = (pltpu.async_copy(out_v.at[b], y_hbm.at[pl.ds(off, _CCH)],
                                    sout[b]),)
        if c + 2 < _CNCH:
            cidx[b] = issue_idx(c + 2)
    wait_all(cout[0])
    if cout[1] is not None:
        wait_all(cout[1])


def _run_combine(ys, sh, p0, p1, w0, w1):
    kern = functools.partial(
        pl.kernel,
        mesh=plsc.VectorSubcoreMesh(core_axis_name="c", subcore_axis_name="s"),
        out_type=jax.ShapeDtypeStruct((T, H), jnp.float32),
        scratch_types=[
            pltpu.VMEM((2, _CCH), jnp.int32),
            pltpu.VMEM((2, _CCH), jnp.int32),
            pltpu.VMEM((2, _CCH, H), jnp.float32),
            pltpu.VMEM((2, _CCH, H), jnp.float32),
            pltpu.VMEM((2, _CCH, H), jnp.float32),
            pltpu.VMEM((2, _CCH, 16), jnp.float32),
            pltpu.VMEM((2, _CCH, 16), jnp.float32),
            pltpu.VMEM((2, _CCH, H), jnp.float32),
            pltpu.SemaphoreType.DMA,
            pltpu.SemaphoreType.DMA,
            pltpu.SemaphoreType.DMA,
            pltpu.SemaphoreType.DMA,
            pltpu.SemaphoreType.DMA,
            pltpu.SemaphoreType.DMA,
        ],
    )(_combine_body)
    return kern(ys, sh, p0, p1, w0, w1)


def kernel(hidden_states, gate_weight, Wg, Wu, Wd, Wsg, Wsu, Wsd):
    bsz, seq_len, h = hidden_states.shape
    x = hidden_states.reshape(-1, h)

    dest4, meta, w0, w1 = _run_router(x, gate_weight)
    dest = dest4.reshape(A)
    tile_e = meta[0, :NT]
    na = meta[0, NT:NT + 1]
    xs = _run_dispatch(x, dest)
    sh = _run_shared(x, Wsg, Wsu, Wsd)
    ys = _run_grouped(tile_e, na, xs, Wg, Wu, Wd)
    y = _run_combine(ys, sh, dest[:T], dest[T:], w0, w1)
    return y.reshape(bsz, seq_len, h)


# speedup vs baseline: 1.0532x; 1.0532x over previous
import functools

import jax
import jax.numpy as jnp
from jax import lax
from jax.experimental import pallas as pl
from jax.experimental.pallas import tpu as pltpu
from jax.experimental.pallas import tpu_sc as plsc

T = 2048
H = 1024
E = 8
F = 1024
NSH = 2
A = 2 * T
TILE = 256
AMAX = A + E * TILE
NT = AMAX // TILE

NC = 2
NS = 16
NW = NC * NS

_DN = (((1,), (1,)), ((), ()))


def _router_kernel(x_ref, gate_ref, dest_ref, meta_ref, w0_ref, w1_ref):
    x = x_ref[...]
    logits = lax.dot_general(x, gate_ref[...], _DN,
                             preferred_element_type=jnp.float32)
    mx = jnp.max(logits, axis=1, keepdims=True)
    p = jnp.exp(logits - mx)
    p = p / jnp.sum(p, axis=1, keepdims=True)
    cols = lax.broadcasted_iota(jnp.int32, p.shape, 1)
    i1 = jnp.argmax(p, axis=1)
    m1 = jnp.max(p, axis=1)
    oh1 = cols == i1[:, None]
    p2 = jnp.where(oh1, -1.0, p)
    i2 = jnp.argmax(p2, axis=1)
    m2 = jnp.max(p2, axis=1)
    denom = m1 + m2 + 1e-20
    w0_ref[...] = jnp.broadcast_to((m1 / denom)[:, None], (T, 16))
    w1_ref[...] = jnp.broadcast_to((m2 / denom)[:, None], (T, 16))

    ohA = (cols == i1[:, None]).astype(jnp.float32)
    ohB = (cols == i2[:, None]).astype(jnp.float32)
    counts = jnp.sum(ohA, axis=0) + jnp.sum(ohB, axis=0)
    pc = jnp.floor((counts + (TILE - 1)) * (1.0 / TILE)) * TILE
    tri8 = (lax.broadcasted_iota(jnp.int32, (E, E), 0)
            > lax.broadcasted_iota(jnp.int32, (E, E), 1)).astype(jnp.float32)
    ps = lax.dot_general(pc[None, :], tri8, _DN,
                         preferred_element_type=jnp.float32)[0]
    total = jnp.sum(pc)
    na = total * (1.0 / TILE)

    ti = lax.broadcasted_iota(jnp.int32, (128, E), 0).astype(jnp.float32)
    ti = jnp.minimum(ti, na - 1.0) * float(TILE)
    te = jnp.sum((ps[None, :] <= ti).astype(jnp.float32), axis=1) - 1.0
    lane = lax.broadcasted_iota(jnp.int32, (128,), 0)
    meta = jnp.where(lane == NT, na, te)
    meta_ref[...] = meta.astype(jnp.int32)[None, :]

    CH = T // 2
    rows = lax.broadcasted_iota(jnp.int32, (CH, CH), 0)
    colsq = lax.broadcasted_iota(jnp.int32, (CH, CH), 1)
    lt = (rows >= colsq).astype(jnp.float32)
    carry = jnp.zeros((E,), jnp.float32)
    for c in range(4):
        oh = (ohA, ohA, ohB, ohB)[c]
        oh = oh[(c % 2) * CH:(c % 2) * CH + CH]
        cum = lax.dot_general(lt, oh, (((1,), (0,)), ((), ())),
                              preferred_element_type=jnp.float32)
        cum = cum + carry[None, :]
        carry = carry + jnp.sum(oh, axis=0)
        rank = jnp.sum(oh * cum, axis=1) - 1.0
        start = jnp.sum(oh * ps[None, :], axis=1)
        dest_ref[pl.ds(c * CH, CH)] = (start + rank).astype(jnp.int32)


def _run_router(x, gate_weight):
    return pl.pallas_call(
        _router_kernel,
        out_shape=(
            jax.ShapeDtypeStruct((A,), jnp.int32),
            jax.ShapeDtypeStruct((1, 128), jnp.int32),
            jax.ShapeDtypeStruct((T, 16), jnp.float32),
            jax.ShapeDtypeStruct((T, 16), jnp.float32),
        ),
    )(x, gate_weight)


_DCH = 32
_DNCH = (A // NW) // _DCH


def _dispatch_body(x_hbm, dest_hbm, xs_hbm, idx_v, rows_v,
                   si0, si1, so0, so1):
    wid = lax.axis_index("s") * NC + lax.axis_index("c")
    apw = A // NW
    base = wid * apw
    tok = jnp.where(base >= T, base - T, base)
    sin = (si0, si1)
    sout = (so0, so1)

    def issue_in(c):
        b = c % 2
        off = c * _DCH
        ci = pltpu.async_copy(dest_hbm.at[pl.ds(base + off, _DCH)],
                              idx_v.at[b], sin[b])
        cr = pltpu.async_copy(x_hbm.at[pl.ds(tok + off, _DCH)],
                              rows_v.at[b], sin[b])
        return ci, cr

    cin = [issue_in(0), issue_in(1)]
    for c in range(_DNCH):
        b = c % 2
        ci, cr = cin[b]
        ci.wait()
        cr.wait()
        cs = pltpu.async_copy(rows_v.at[b], xs_hbm.at[idx_v.at[b]], sout[b])
        if c + 2 < _DNCH:
            cs.wait()
            cin[b] = issue_in(c + 2)
        else:
            cs.wait()


def _run_dispatch(x, dest):
    kern = functools.partial(
        pl.kernel,
        mesh=plsc.VectorSubcoreMesh(core_axis_name="c", subcore_axis_name="s"),
        out_type=jax.ShapeDtypeStruct((AMAX, H), jnp.float32),
        scratch_types=[
            pltpu.VMEM((2, _DCH), jnp.int32),
            pltpu.VMEM((2, _DCH, H), jnp.float32),
            pltpu.SemaphoreType.DMA,
            pltpu.SemaphoreType.DMA,
            pltpu.SemaphoreType.DMA,
            pltpu.SemaphoreType.DMA,
        ],
    )(_dispatch_body)
    return kern(x, dest)


def _grouped_kernel(tile_e_ref, na_ref, xs_ref, wg_ref, wu_ref, wd_ref,
                    ys_ref):
    i = pl.program_id(0)

    @pl.when(i < na_ref[0])
    def _():
        x = xs_ref[...]
        g = lax.dot_general(x, wg_ref[0].astype(jnp.bfloat16), _DN,
                            preferred_element_type=jnp.float32)
        u = lax.dot_general(x, wu_ref[0].astype(jnp.bfloat16), _DN,
                            preferred_element_type=jnp.float32)
        h = (g * jax.nn.sigmoid(g)) * u
        ys_ref[...] = lax.dot_general(h.astype(jnp.bfloat16),
                                      wd_ref[0].astype(jnp.bfloat16), _DN,
                                      preferred_element_type=jnp.float32)


def _run_grouped(tile_e, na, xs, Wg, Wu, Wd):
    grid_spec = pltpu.PrefetchScalarGridSpec(
        num_scalar_prefetch=2,
        grid=(NT,),
        in_specs=[
            pl.BlockSpec((TILE, H),
                         lambda i, te, na: (jnp.minimum(i, na[0] - 1), 0)),
            pl.BlockSpec((1, F, H), lambda i, te, na: (te[i], 0, 0)),
            pl.BlockSpec((1, F, H), lambda i, te, na: (te[i], 0, 0)),
            pl.BlockSpec((1, H, F), lambda i, te, na: (te[i], 0, 0)),
        ],
        out_specs=pl.BlockSpec(
            (TILE, H), lambda i, te, na: (jnp.minimum(i, na[0] - 1), 0)),
    )
    return pl.pallas_call(
        _grouped_kernel,
        grid_spec=grid_spec,
        out_shape=jax.ShapeDtypeStruct((AMAX, H), jnp.float32),
    )(tile_e, na, xs, Wg, Wu, Wd)


_TMS = 1024


def _shared_kernel(x_ref, wsg_ref, wsu_ref, wsd_ref, out_ref):
    c = pl.program_id(1)
    x = x_ref[...].astype(jnp.bfloat16)
    g = lax.dot_general(x, wsg_ref[...].astype(jnp.bfloat16), _DN,
                        preferred_element_type=jnp.float32)
    u = lax.dot_general(x, wsu_ref[...].astype(jnp.bfloat16), _DN,
                        preferred_element_type=jnp.float32)
    h = (g * jax.nn.sigmoid(g)) * u
    y = lax.dot_general(h.astype(jnp.bfloat16),
                        wsd_ref[...].astype(jnp.bfloat16), _DN,
                        preferred_element_type=jnp.float32)

    @pl.when(c == 0)
    def _():
        out_ref[...] = y

    @pl.when(c == 1)
    def _():
        out_ref[...] += y


def _run_shared(x, Wsg, Wsu, Wsd):
    return pl.pallas_call(
        _shared_kernel,
        grid=(T // _TMS, NSH),
        in_specs=[
            pl.BlockSpec((_TMS, H), lambda t, c: (t, 0)),
            pl.BlockSpec((F, H), lambda t, c: (c, 0)),
            pl.BlockSpec((F, H), lambda t, c: (c, 0)),
            pl.BlockSpec((H, F), lambda t, c: (0, c)),
        ],
        out_specs=pl.BlockSpec((_TMS, H), lambda t, c: (t, 0)),
        out_shape=jax.ShapeDtypeStruct((T, H), jnp.float32),
    )(x, Wsg, Wsu, Wsd)


_CCH = 8
_CNCH = (T // NW) // _CCH


def _combine_body(ys_hbm, sh_hbm, dest_hbm, w0_hbm, w1_hbm, y_hbm,
                  p0_v, p1_v, r0_v, r1_v, sh_v, w0_v, w1_v, out_v,
                  sx0, sx1, sg0, sg1, so0, so1):
    wid = lax.axis_index("s") * NC + lax.axis_index("c")
    tpw = T // NW
    base = wid * tpw
    sidx = (sx0, sx1)
    sgat = (sg0, sg1)
    sout = (so0, so1)

    def issue_idx(c):
        b = c % 2
        off = base + c * _CCH
        return (pltpu.async_copy(dest_hbm.at[pl.ds(off, _CCH)],
                                 p0_v.at[b], sidx[b]),
                pltpu.async_copy(dest_hbm.at[pl.ds(T + off, _CCH)],
                                 p1_v.at[b], sidx[b]))

    def issue_gather(c):
        b = c % 2
        off = base + c * _CCH
        return (pltpu.async_copy(ys_hbm.at[p0_v.at[b]], r0_v.at[b], sgat[b]),
                pltpu.async_copy(ys_hbm.at[p1_v.at[b]], r1_v.at[b], sgat[b]),
                pltpu.async_copy(sh_hbm.at[pl.ds(off, _CCH)],
                                 sh_v.at[b], sgat[b]),
                pltpu.async_copy(w0_hbm.at[pl.ds(off, _CCH)],
                                 w0_v.at[b], sgat[b]),
                pltpu.async_copy(w1_hbm.at[pl.ds(off, _CCH)],
                                 w1_v.at[b], sgat[b]))

    def wait_all(cps):
        for cp in cps:
            cp.wait()

    cidx = [issue_idx(0), None]
    wait_all(cidx[0])
    cgat = [issue_gather(0), None]
    cidx[1] = issue_idx(1)
    cout = [None, None]
    for c in range(_CNCH):
        b = c % 2
        nb = (c + 1) % 2
        if c + 1 < _CNCH:
            wait_all(cidx[nb])
            cgat[nb] = issue_gather(c + 1)
            if c + 2 < _CNCH:
                cidx[b] = None
        wait_all(cgat[b])
        if cout[b] is not None:
            wait_all(cout[b])

        def body(j, _):
            w0 = w0_v[b, j, :]
            w1 = w1_v[b, j, :]
            for cc in range(H // 16):
                s = pl.ds(cc * 16, 16)
                out_v[b, j, s] = (sh_v[b, j, s] + w0 * r0_v[b, j, s]
                                  + w1 * r1_v[b, j, s])
            return 0

        lax.fori_loop(0, _CCH, body, 0)
        off = base + c * _CCH
        cout[b] = (pltpu.async_copy(out_v.at[b], y_hbm.at[pl.ds(off, _CCH)],
                                    sout[b]),)
        if c + 2 < _CNCH:
            cidx[b] = issue_idx(c + 2)
    wait_all(cout[0])
    if cout[1] is not None:
        wait_all(cout[1])


def _run_combine(ys, sh, dest, w0, w1):
    kern = functools.partial(
        pl.kernel,
        mesh=plsc.VectorSubcoreMesh(core_axis_name="c", subcore_axis_name="s"),
        out_type=jax.ShapeDtypeStruct((T, H), jnp.float32),
        scratch_types=[
            pltpu.VMEM((2, _CCH), jnp.int32),
            pltpu.VMEM((2, _CCH), jnp.int32),
            pltpu.VMEM((2, _CCH, H), jnp.float32),
            pltpu.VMEM((2, _CCH, H), jnp.float32),
            pltpu.VMEM((2, _CCH, H), jnp.float32),
            pltpu.VMEM((2, _CCH, 16), jnp.float32),
            pltpu.VMEM((2, _CCH, 16), jnp.float32),
            pltpu.VMEM((2, _CCH, H), jnp.float32),
            pltpu.SemaphoreType.DMA,
            pltpu.SemaphoreType.DMA,
            pltpu.SemaphoreType.DMA,
            pltpu.SemaphoreType.DMA,
            pltpu.SemaphoreType.DMA,
            pltpu.SemaphoreType.DMA,
        ],
    )(_combine_body)
    return kern(ys, sh, dest, w0, w1)


def kernel(hidden_states, gate_weight, Wg, Wu, Wd, Wsg, Wsu, Wsd):
    bsz, seq_len, h = hidden_states.shape
    x = hidden_states.reshape(-1, h)

    dest, meta, w0, w1 = _run_router(x, gate_weight)
    tile_e = meta[0, :NT]
    na = meta[0, NT:NT + 1]
    xs = _run_dispatch(x, dest)
    sh = _run_shared(x, Wsg, Wsu, Wsd)
    ys = _run_grouped(tile_e, na, xs, Wg, Wu, Wd)
    y = _run_combine(ys, sh, dest, w0, w1)
    return y.reshape(bsz, seq_len, h)


# speedup vs baseline: 1.1313x; 1.0742x over previous
import functools

import jax
import jax.numpy as jnp
from jax import lax
from jax.experimental import pallas as pl
from jax.experimental.pallas import tpu as pltpu
from jax.experimental.pallas import tpu_sc as plsc

T = 2048
H = 1024
E = 8
F = 1024
NSH = 2
A = 2 * T
TILE = 512
AMAX = A + E * TILE
NT = AMAX // TILE

NC = 2
NS = 16
NW = NC * NS

_DN = (((1,), (1,)), ((), ()))


def _router_kernel(x_ref, gate_ref, dest_ref, meta_ref, w0_ref, w1_ref):
    x = x_ref[...]
    logits = lax.dot_general(x, gate_ref[...], _DN,
                             preferred_element_type=jnp.float32)
    mx = jnp.max(logits, axis=1, keepdims=True)
    p = jnp.exp(logits - mx)
    p = p / jnp.sum(p, axis=1, keepdims=True)
    cols = lax.broadcasted_iota(jnp.int32, p.shape, 1)
    i1 = jnp.argmax(p, axis=1)
    m1 = jnp.max(p, axis=1)
    oh1 = cols == i1[:, None]
    p2 = jnp.where(oh1, -1.0, p)
    i2 = jnp.argmax(p2, axis=1)
    m2 = jnp.max(p2, axis=1)
    denom = m1 + m2 + 1e-20
    w0_ref[...] = jnp.broadcast_to((m1 / denom)[:, None], (T, 16))
    w1_ref[...] = jnp.broadcast_to((m2 / denom)[:, None], (T, 16))

    ohA = (cols == i1[:, None]).astype(jnp.float32)
    ohB = (cols == i2[:, None]).astype(jnp.float32)
    counts = jnp.sum(ohA, axis=0) + jnp.sum(ohB, axis=0)
    pc = jnp.floor((counts + (TILE - 1)) * (1.0 / TILE)) * TILE
    tri8 = (lax.broadcasted_iota(jnp.int32, (E, E), 0)
            > lax.broadcasted_iota(jnp.int32, (E, E), 1)).astype(jnp.float32)
    ps = lax.dot_general(pc[None, :], tri8, _DN,
                         preferred_element_type=jnp.float32)[0]
    total = jnp.sum(pc)
    na = total * (1.0 / TILE)

    ti = lax.broadcasted_iota(jnp.int32, (128, E), 0).astype(jnp.float32)
    ti = jnp.minimum(ti, na - 1.0) * float(TILE)
    te = jnp.sum((ps[None, :] <= ti).astype(jnp.float32), axis=1) - 1.0
    lane = lax.broadcasted_iota(jnp.int32, (128,), 0)
    meta = jnp.where(lane == NT, na, te)
    meta_ref[...] = meta.astype(jnp.int32)[None, :]

    CH = T // 2
    rows = lax.broadcasted_iota(jnp.int32, (CH, CH), 0)
    colsq = lax.broadcasted_iota(jnp.int32, (CH, CH), 1)
    lt = (rows >= colsq).astype(jnp.float32)
    carry = jnp.zeros((E,), jnp.float32)
    for c in range(4):
        oh = (ohA, ohA, ohB, ohB)[c]
        oh = oh[(c % 2) * CH:(c % 2) * CH + CH]
        cum = lax.dot_general(lt, oh, (((1,), (0,)), ((), ())),
                              preferred_element_type=jnp.float32)
        cum = cum + carry[None, :]
        carry = carry + jnp.sum(oh, axis=0)
        rank = jnp.sum(oh * cum, axis=1) - 1.0
        start = jnp.sum(oh * ps[None, :], axis=1)
        dest_ref[pl.ds(c * CH, CH)] = (start + rank).astype(jnp.int32)


def _run_router(x, gate_weight):
    return pl.pallas_call(
        _router_kernel,
        out_shape=(
            jax.ShapeDtypeStruct((A,), jnp.int32),
            jax.ShapeDtypeStruct((1, 128), jnp.int32),
            jax.ShapeDtypeStruct((T, 16), jnp.float32),
            jax.ShapeDtypeStruct((T, 16), jnp.float32),
        ),
    )(x, gate_weight)


_DCH = 32
_DNCH = (A // NW) // _DCH


def _dispatch_body(x_hbm, dest_hbm, xs_hbm, idx_v, rows_v,
                   si0, si1, so0, so1):
    wid = lax.axis_index("s") * NC + lax.axis_index("c")
    apw = A // NW
    base = wid * apw
    tok = jnp.where(base >= T, base - T, base)
    sin = (si0, si1)
    sout = (so0, so1)

    def issue_in(c):
        b = c % 2
        off = c * _DCH
        ci = pltpu.async_copy(dest_hbm.at[pl.ds(base + off, _DCH)],
                              idx_v.at[b], sin[b])
        cr = pltpu.async_copy(x_hbm.at[pl.ds(tok + off, _DCH)],
                              rows_v.at[b], sin[b])
        return ci, cr

    cin = [issue_in(0), issue_in(1)]
    for c in range(_DNCH):
        b = c % 2
        ci, cr = cin[b]
        ci.wait()
        cr.wait()
        cs = pltpu.async_copy(rows_v.at[b], xs_hbm.at[idx_v.at[b]], sout[b])
        if c + 2 < _DNCH:
            cs.wait()
            cin[b] = issue_in(c + 2)
        else:
            cs.wait()


def _run_dispatch(x, dest):
    kern = functools.partial(
        pl.kernel,
        mesh=plsc.VectorSubcoreMesh(core_axis_name="c", subcore_axis_name="s"),
        out_type=jax.ShapeDtypeStruct((AMAX, H), jnp.float32),
        scratch_types=[
            pltpu.VMEM((2, _DCH), jnp.int32),
            pltpu.VMEM((2, _DCH, H), jnp.float32),
            pltpu.SemaphoreType.DMA,
            pltpu.SemaphoreType.DMA,
            pltpu.SemaphoreType.DMA,
            pltpu.SemaphoreType.DMA,
        ],
    )(_dispatch_body)
    return kern(x, dest)


def _grouped_kernel(tile_e_ref, na_ref, xs_ref, wg_ref, wu_ref, wd_ref,
                    ys_ref):
    i = pl.program_id(0)

    @pl.when(i < na_ref[0])
    def _():
        x = xs_ref[...]
        g = lax.dot_general(x, wg_ref[0].astype(jnp.bfloat16), _DN,
                            preferred_element_type=jnp.float32)
        u = lax.dot_general(x, wu_ref[0].astype(jnp.bfloat16), _DN,
                            preferred_element_type=jnp.float32)
        h = (g * jax.nn.sigmoid(g)) * u
        ys_ref[...] = lax.dot_general(h.astype(jnp.bfloat16),
                                      wd_ref[0].astype(jnp.bfloat16), _DN,
                                      preferred_element_type=jnp.float32)


def _run_grouped(tile_e, na, xs, Wg, Wu, Wd):
    grid_spec = pltpu.PrefetchScalarGridSpec(
        num_scalar_prefetch=2,
        grid=(NT,),
        in_specs=[
            pl.BlockSpec((TILE, H),
                         lambda i, te, na: (jnp.minimum(i, na[0] - 1), 0)),
            pl.BlockSpec((1, F, H), lambda i, te, na: (te[i], 0, 0)),
            pl.BlockSpec((1, F, H), lambda i, te, na: (te[i], 0, 0)),
            pl.BlockSpec((1, H, F), lambda i, te, na: (te[i], 0, 0)),
        ],
        out_specs=pl.BlockSpec(
            (TILE, H), lambda i, te, na: (jnp.minimum(i, na[0] - 1), 0)),
    )
    return pl.pallas_call(
        _grouped_kernel,
        grid_spec=grid_spec,
        out_shape=jax.ShapeDtypeStruct((AMAX, H), jnp.float32),
    )(tile_e, na, xs, Wg, Wu, Wd)


_TMS = 1024


def _shared_kernel(x_ref, wsg_ref, wsu_ref, wsd_ref, out_ref):
    c = pl.program_id(1)
    x = x_ref[...].astype(jnp.bfloat16)
    g = lax.dot_general(x, wsg_ref[...].astype(jnp.bfloat16), _DN,
                        preferred_element_type=jnp.float32)
    u = lax.dot_general(x, wsu_ref[...].astype(jnp.bfloat16), _DN,
                        preferred_element_type=jnp.float32)
    h = (g * jax.nn.sigmoid(g)) * u
    y = lax.dot_general(h.astype(jnp.bfloat16),
                        wsd_ref[...].astype(jnp.bfloat16), _DN,
                        preferred_element_type=jnp.float32)

    @pl.when(c == 0)
    def _():
        out_ref[...] = y

    @pl.when(c == 1)
    def _():
        out_ref[...] += y


def _run_shared(x, Wsg, Wsu, Wsd):
    return pl.pallas_call(
        _shared_kernel,
        grid=(T // _TMS, NSH),
        in_specs=[
            pl.BlockSpec((_TMS, H), lambda t, c: (t, 0)),
            pl.BlockSpec((F, H), lambda t, c: (c, 0)),
            pl.BlockSpec((F, H), lambda t, c: (c, 0)),
            pl.BlockSpec((H, F), lambda t, c: (0, c)),
        ],
        out_specs=pl.BlockSpec((_TMS, H), lambda t, c: (t, 0)),
        out_shape=jax.ShapeDtypeStruct((T, H), jnp.float32),
    )(x, Wsg, Wsu, Wsd)


_CCH = 8
_CNCH = (T // NW) // _CCH


def _combine_body(ys_hbm, sh_hbm, dest_hbm, w0_hbm, w1_hbm, y_hbm,
                  p0_v, p1_v, r0_v, r1_v, sh_v, w0_v, w1_v, out_v,
                  sx0, sx1, sg0, sg1, so0, so1):
    wid = lax.axis_index("s") * NC + lax.axis_index("c")
    tpw = T // NW
    base = wid * tpw
    sidx = (sx0, sx1)
    sgat = (sg0, sg1)
    sout = (so0, so1)

    def issue_idx(c):
        b = c % 2
        off = base + c * _CCH
        return (pltpu.async_copy(dest_hbm.at[pl.ds(off, _CCH)],
                                 p0_v.at[b], sidx[b]),
                pltpu.async_copy(dest_hbm.at[pl.ds(T + off, _CCH)],
                                 p1_v.at[b], sidx[b]))

    def issue_gather(c):
        b = c % 2
        off = base + c * _CCH
        return (pltpu.async_copy(ys_hbm.at[p0_v.at[b]], r0_v.at[b], sgat[b]),
                pltpu.async_copy(ys_hbm.at[p1_v.at[b]], r1_v.at[b], sgat[b]),
                pltpu.async_copy(sh_hbm.at[pl.ds(off, _CCH)],
                                 sh_v.at[b], sgat[b]),
                pltpu.async_copy(w0_hbm.at[pl.ds(off, _CCH)],
                                 w0_v.at[b], sgat[b]),
                pltpu.async_copy(w1_hbm.at[pl.ds(off, _CCH)],
                                 w1_v.at[b], sgat[b]))

    def wait_all(cps):
        for cp in cps:
            cp.wait()

    cidx = [issue_idx(0), None]
    wait_all(cidx[0])
    cgat = [issue_gather(0), None]
    cidx[1] = issue_idx(1)
    cout = [None, None]
    for c in range(_CNCH):
        b = c % 2
        nb = (c + 1) % 2
        if c + 1 < _CNCH:
            wait_all(cidx[nb])
            cgat[nb] = issue_gather(c + 1)
            if c + 2 < _CNCH:
                cidx[b] = None
        wait_all(cgat[b])
        if cout[b] is not None:
            wait_all(cout[b])

        def body(j, _):
            w0 = w0_v[b, j, :]
            w1 = w1_v[b, j, :]
            for cc in range(H // 16):
                s = pl.ds(cc * 16, 16)
                out_v[b, j, s] = (sh_v[b, j, s] + w0 * r0_v[b, j, s]
                                  + w1 * r1_v[b, j, s])
            return 0

        lax.fori_loop(0, _CCH, body, 0)
        off = base + c * _CCH
        cout[b] = (pltpu.async_copy(out_v.at[b], y_hbm.at[pl.ds(off, _CCH)],
                                    sout[b]),)
        if c + 2 < _CNCH:
            cidx[b] = issue_idx(c + 2)
    wait_all(cout[0])
    if cout[1] is not None:
        wait_all(cout[1])


def _run_combine(ys, sh, dest, w0, w1):
    kern = functools.partial(
        pl.kernel,
        mesh=plsc.VectorSubcoreMesh(core_axis_name="c", subcore_axis_name="s"),
        out_type=jax.ShapeDtypeStruct((T, H), jnp.float32),
        scratch_types=[
            pltpu.VMEM((2, _CCH), jnp.int32),
            pltpu.VMEM((2, _CCH), jnp.int32),
            pltpu.VMEM((2, _CCH, H), jnp.float32),
            pltpu.VMEM((2, _CCH, H), jnp.float32),
            pltpu.VMEM((2, _CCH, H), jnp.float32),
            pltpu.VMEM((2, _CCH, 16), jnp.float32),
            pltpu.VMEM((2, _CCH, 16), jnp.float32),
            pltpu.VMEM((2, _CCH, H), jnp.float32),
            pltpu.SemaphoreType.DMA,
            pltpu.SemaphoreType.DMA,
            pltpu.SemaphoreType.DMA,
            pltpu.SemaphoreType.DMA,
            pltpu.SemaphoreType.DMA,
            pltpu.SemaphoreType.DMA,
        ],
    )(_combine_body)
    return kern(ys, sh, dest, w0, w1)


def kernel(hidden_states, gate_weight, Wg, Wu, Wd, Wsg, Wsu, Wsd):
    bsz, seq_len, h = hidden_states.shape
    x = hidden_states.reshape(-1, h)

    dest, meta, w0, w1 = _run_router(x, gate_weight)
    tile_e = meta[0, :NT]
    na = meta[0, NT:NT + 1]
    xs = _run_dispatch(x, dest)
    sh = _run_shared(x, Wsg, Wsu, Wsd)
    ys = _run_grouped(tile_e, na, xs, Wg, Wu, Wd)
    y = _run_combine(ys, sh, dest, w0, w1)
    return y.reshape(bsz, seq_len, h)
